# baseline (speedup 1.0000x reference)
import jax
import jax.numpy as jnp
from jax.experimental import pallas as pl
from jax.experimental.pallas import tpu as pltpu

_ROWS = 8
_BN = 1024


def _chamfer_body(thr_ref, p_ref, gb_ref, out_ref, bwd_scr):
    m = p_ref.shape[0]
    n = gb_ref.shape[2]
    nj = n // _BN
    nr = m // _ROWS
    inf = jnp.float32(jnp.inf)

    bwd_scr[:, :] = jnp.full((_ROWS, n), inf, jnp.float32)
    t0 = thr_ref[0]
    t1 = thr_ref[1]

    def rbody(r, carry):
        fsum, c0, c1 = carry
        pch = p_ref[pl.ds(r * _ROWS, _ROWS), :]
        p0 = pch[:, 0:1]
        p1 = pch[:, 1:2]
        p2 = pch[:, 2:3]
        facc = jnp.full((_ROWS, _BN), inf, jnp.float32)
        for j in range(nj):
            sl = pl.ds(j * _BN, _BN)
            d0 = p0 - gb_ref[0, :, sl]
            d1 = p1 - gb_ref[1, :, sl]
            d2c = p2 - gb_ref[2, :, sl]
            d2 = d0 * d0 + d1 * d1 + d2c * d2c
            facc = jnp.minimum(facc, d2)
            bwd_scr[:, sl] = jnp.minimum(bwd_scr[:, sl], d2)
        fdist = jnp.sqrt(jnp.min(facc, axis=1, keepdims=True))
        fsum = fsum + fdist
        c0 = c0 + (fdist <= t0).astype(jnp.float32)
        c1 = c1 + (fdist <= t1).astype(jnp.float32)
        return fsum, c0, c1

    zero = jnp.zeros((_ROWS, 1), jnp.float32)
    fsum, c0, c1 = jax.lax.fori_loop(0, nr, rbody, (zero, zero, zero))
    fsum_s = jnp.sum(fsum)
    fc0 = jnp.sum(c0)
    fc1 = jnp.sum(c1)

    bdist = jnp.sqrt(jnp.min(bwd_scr[:, :], axis=0, keepdims=True))
    bsum_s = jnp.sum(bdist)
    bc0 = jnp.sum((bdist <= t0).astype(jnp.float32))
    bc1 = jnp.sum((bdist <= t1).astype(jnp.float32))

    mf = jnp.float32(m)
    nf = jnp.float32(n)
    cd = fsum_s / mf * 0.5 + bsum_s / nf * 0.5

    def fsc(fc, bc):
        prec = 100.0 / mf * fc
        rec = 100.0 / nf * bc
        return 2.0 * prec * rec / (prec + rec + 1e-8)

    f0 = fsc(fc0, bc0)
    f1 = fsc(fc1, bc1)
    lane = jax.lax.broadcasted_iota(jnp.int32, (1, 128), 1)
    out_ref[:, :] = jnp.where(
        lane == 0, cd, jnp.where(lane == 1, f0, jnp.where(lane == 2, f1, 0.0))
    ).astype(jnp.float32)


def _chamfer(p, gb, threshes, interpret=False):
    return pl.pallas_call(
        _chamfer_body,
        out_shape=jax.ShapeDtypeStruct((1, 128), jnp.float32),
        in_specs=[
            pl.BlockSpec(memory_space=pltpu.SMEM),
            pl.BlockSpec(memory_space=pltpu.VMEM),
            pl.BlockSpec(memory_space=pltpu.VMEM),
        ],
        out_specs=pl.BlockSpec(memory_space=pltpu.VMEM),
        scratch_shapes=[pltpu.VMEM((_ROWS, gb.shape[2]), jnp.float32)],
        interpret=interpret,
    )(threshes, p, gb)


def kernel(predict_pc, gt_pc, threshes):
    p = jnp.transpose(predict_pc[0], (1, 0))
    g = gt_pc[0]
    gb = jnp.broadcast_to(g[:, None, :], (3, _ROWS, g.shape[1]))
    out = _chamfer(p, gb, threshes)
    return out[0, :3]


# speedup vs baseline: 1.0293x; 1.0293x over previous
import jax
import jax.numpy as jnp
from jax.experimental import pallas as pl
from jax.experimental.pallas import tpu as pltpu

_ROWS = 8
_BN = 1024
_INF_BITS = 0x7F800000


def _bitcast_u32(x):
    return jax.lax.bitcast_convert_type(x, jnp.int32)


def _bitcast_f32(x):
    return jax.lax.bitcast_convert_type(x, jnp.float32)


def _chamfer_body(thr_ref, p_ref, gb_ref, gsq_ref, out_ref, bwd_scr):
    m = p_ref.shape[0]
    n = gb_ref.shape[2]
    nj = n // _BN
    nr = m // _ROWS

    bwd_scr[:, :] = jnp.full((_ROWS, n), _INF_BITS, jnp.int32)
    t0 = thr_ref[0]
    t1 = thr_ref[1]

    def rbody(r, carry):
        fsum, c0, c1 = carry
        pch = p_ref[pl.ds(r * _ROWS, _ROWS), :]
        p0 = pch[:, 0:1]
        p1 = pch[:, 1:2]
        p2 = pch[:, 2:3]
        psq = pch[:, 3:4]
        facc = jnp.full((_ROWS, _BN), _INF_BITS, jnp.int32)
        for j in range(nj):
            sl = pl.ds(j * _BN, _BN)
            d2 = (psq + gsq_ref[:, sl]) + (
                p0 * gb_ref[0, :, sl]
                + p1 * gb_ref[1, :, sl]
                + p2 * gb_ref[2, :, sl]
            )
            d2u = _bitcast_u32(d2)
            facc = jnp.minimum(facc, d2u)
            bwd_scr[:, sl] = jnp.minimum(bwd_scr[:, sl], d2u)
        rmin = jnp.min(_bitcast_f32(facc), axis=1, keepdims=True)
        fdist = jnp.sqrt(jnp.maximum(rmin, 0.0))
        fsum = fsum + fdist
        c0 = c0 + (fdist <= t0).astype(jnp.float32)
        c1 = c1 + (fdist <= t1).astype(jnp.float32)
        return fsum, c0, c1

    zero = jnp.zeros((_ROWS, 1), jnp.float32)
    fsum, c0, c1 = jax.lax.fori_loop(0, nr, rbody, (zero, zero, zero))
    fsum_s = jnp.sum(fsum)
    fc0 = jnp.sum(c0)
    fc1 = jnp.sum(c1)

    bmin = jnp.min(_bitcast_f32(bwd_scr[:, :]), axis=0, keepdims=True)
    bdist = jnp.sqrt(jnp.maximum(bmin, 0.0))
    bsum_s = jnp.sum(bdist)
    bc0 = jnp.sum((bdist <= t0).astype(jnp.float32))
    bc1 = jnp.sum((bdist <= t1).astype(jnp.float32))

    mf = jnp.float32(m)
    nf = jnp.float32(n)
    cd = fsum_s / mf * 0.5 + bsum_s / nf * 0.5

    def fsc(fc, bc):
        prec = 100.0 / mf * fc
        rec = 100.0 / nf * bc
        return 2.0 * prec * rec / (prec + rec + 1e-8)

    f0 = fsc(fc0, bc0)
    f1 = fsc(fc1, bc1)
    lane = jax.lax.broadcasted_iota(jnp.int32, (1, 128), 1)
    out_ref[:, :] = jnp.where(
        lane == 0, cd, jnp.where(lane == 1, f0, jnp.where(lane == 2, f1, 0.0))
    ).astype(jnp.float32)


def _chamfer(p4, gb, gsqb, threshes, interpret=False):
    return pl.pallas_call(
        _chamfer_body,
        out_shape=jax.ShapeDtypeStruct((1, 128), jnp.float32),
        in_specs=[
            pl.BlockSpec(memory_space=pltpu.SMEM),
            pl.BlockSpec(memory_space=pltpu.VMEM),
            pl.BlockSpec(memory_space=pltpu.VMEM),
            pl.BlockSpec(memory_space=pltpu.VMEM),
        ],
        out_specs=pl.BlockSpec(memory_space=pltpu.VMEM),
        scratch_shapes=[pltpu.VMEM((_ROWS, gb.shape[2]), jnp.int32)],
        interpret=interpret,
    )(threshes, p4, gb, gsqb)


def kernel(predict_pc, gt_pc, threshes):
    p = jnp.transpose(predict_pc[0], (1, 0))
    psq = jnp.sum(p * p, axis=1, keepdims=True)
    p4 = jnp.concatenate([-2.0 * p, psq], axis=1)
    g = gt_pc[0]
    n = g.shape[1]
    gb = jnp.broadcast_to(g[:, None, :], (3, _ROWS, n))
    gsq = jnp.sum(g * g, axis=0, keepdims=True)
    gsqb = jnp.broadcast_to(gsq, (_ROWS, n))
    out = _chamfer(p4, gb, gsqb, threshes)
    return out[0, :3]
